# baseline (speedup 1.0000x reference)
import functools

import jax
import jax.numpy as jnp
from jax import lax
from jax.experimental import pallas as pl
from jax.experimental.pallas import tpu as pltpu
from jax.experimental.pallas import tpu_sc as plsc

_TOP_K = 2
_EXPERTS = 16
_BLK = 1024
_CHUNKS = 2


def _logits_t_kernel(x_ref, w_ref, b_ref, out_ref):
    out_ref[...] = lax.dot_general(
        w_ref[...], x_ref[...],
        (((1,), (1,)), ((), ())),
        preferred_element_type=jnp.float32,
    ) + b_ref[...]


_SC_INFO = plsc.get_sparse_core_info()
_NW = _SC_INFO.num_cores * _SC_INFO.num_subcores
_LANES = _SC_INFO.num_lanes


def _route_sc_body(tpw, logits_hbm, out_hbm, idx_hbm, lbuf, obuf, ibuf):
    wid = lax.axis_index("s") * _SC_INFO.num_cores + lax.axis_index("c")
    base = wid * tpw
    pltpu.sync_copy(logits_hbm.at[:, pl.ds(base, tpw)], lbuf)

    neg_inf = jnp.full((_LANES,), -jnp.inf, dtype=jnp.float32)
    zero = jnp.zeros((_LANES,), dtype=jnp.float32)

    for g in range(tpw // _LANES):
        sl = pl.ds(g * _LANES, _LANES)
        cols = [lbuf[e, sl] for e in range(_EXPERTS)]

        v1 = cols[0]
        idx1 = jnp.zeros((_LANES,), dtype=jnp.int32)
        for e in range(1, _EXPERTS):
            m = cols[e] > v1
            v1 = jnp.where(m, cols[e], v1)
            idx1 = jnp.where(m, jnp.full((_LANES,), e, jnp.int32), idx1)

        cols2 = [jnp.where(idx1 == e, neg_inf, cols[e])
                 for e in range(_EXPERTS)]
        v2 = cols2[0]
        idx2 = jnp.zeros((_LANES,), dtype=jnp.int32)
        for e in range(1, _EXPERTS):
            m = cols2[e] > v2
            v2 = jnp.where(m, cols2[e], v2)
            idx2 = jnp.where(m, jnp.full((_LANES,), e, jnp.int32), idx2)

        t = jnp.exp(v2 - v1)
        denom = 1.0 + t
        p1 = 1.0 / denom
        p2 = t / denom

        for e in range(_EXPERTS):
            obuf[e, sl] = (jnp.where(idx1 == e, p1, zero)
                           + jnp.where(idx2 == e, p2, zero))
        ibuf[0, sl] = idx1
        ibuf[1, sl] = idx2

    pltpu.sync_copy(obuf, out_hbm.at[:, pl.ds(base, tpw)])
    pltpu.sync_copy(ibuf, idx_hbm.at[:, pl.ds(base, tpw)])


def _finalize_kernel(outt_ref, idxt_ref, out_ref, idx_ref):
    out_ref[...] = outt_ref[...].T
    idx_ref[...] = idxt_ref[...].T


def kernel(x, route_W, route_b, noise_W, noise_b):
    del noise_W, noise_b
    tokens = x.shape[0]
    n_embd = x.shape[1]
    bcol = route_b.reshape(_EXPERTS, 1)
    ctok = tokens // _CHUNKS
    tpw = ctok // _NW

    mesh = plsc.VectorSubcoreMesh(core_axis_name="c", subcore_axis_name="s")
    route = functools.partial(
        pl.kernel,
        out_type=[
            jax.ShapeDtypeStruct((_EXPERTS, ctok), jnp.float32),
            jax.ShapeDtypeStruct((_TOP_K, ctok), jnp.int32),
        ],
        mesh=mesh,
        scratch_types=[
            pltpu.VMEM((_EXPERTS, tpw), jnp.float32),
            pltpu.VMEM((_EXPERTS, tpw), jnp.float32),
            pltpu.VMEM((_TOP_K, tpw), jnp.int32),
        ],
    )(functools.partial(_route_sc_body, tpw))

    outs, idxs = [], []
    for c in range(_CHUNKS):
        blk0 = c * (ctok // _BLK)
        logits_t = pl.pallas_call(
            _logits_t_kernel,
            grid=(ctok // _BLK,),
            in_specs=[
                pl.BlockSpec((_BLK, n_embd), lambda i, b=blk0: (b + i, 0)),
                pl.BlockSpec((_EXPERTS, n_embd), lambda i: (0, 0)),
                pl.BlockSpec((_EXPERTS, 1), lambda i: (0, 0)),
            ],
            out_specs=pl.BlockSpec((_EXPERTS, _BLK), lambda i: (0, i)),
            out_shape=jax.ShapeDtypeStruct((_EXPERTS, ctok), jnp.float32),
            compiler_params=pltpu.CompilerParams(
                dimension_semantics=("arbitrary",),
            ),
        )(x, route_W, bcol)

        out_t, idx_t = route(logits_t)

        out_c, idx_c = pl.pallas_call(
            _finalize_kernel,
            grid=(ctok // _BLK,),
            in_specs=[
                pl.BlockSpec((_EXPERTS, _BLK), lambda i: (0, i)),
                pl.BlockSpec((_TOP_K, _BLK), lambda i: (0, i)),
            ],
            out_specs=[
                pl.BlockSpec((_BLK, _EXPERTS), lambda i: (i, 0)),
                pl.BlockSpec((_BLK, _TOP_K), lambda i: (i, 0)),
            ],
            out_shape=[
                jax.ShapeDtypeStruct((ctok, _EXPERTS), jnp.float32),
                jax.ShapeDtypeStruct((ctok, _TOP_K), jnp.int32),
            ],
            compiler_params=pltpu.CompilerParams(
                dimension_semantics=("arbitrary",),
            ),
        )(out_t, idx_t)
        outs.append(out_c)
        idxs.append(idx_c)

    return (jnp.concatenate(outs, axis=0), jnp.concatenate(idxs, axis=0))


# speedup vs baseline: 1.0168x; 1.0168x over previous
import functools

import jax
import jax.numpy as jnp
from jax import lax
from jax.experimental import pallas as pl
from jax.experimental.pallas import tpu as pltpu
from jax.experimental.pallas import tpu_sc as plsc

_TOP_K = 2
_EXPERTS = 16
_BLK = 1024
_CHUNKS = 1


def _logits_t_kernel(x_ref, w_ref, b_ref, out_ref):
    out_ref[...] = lax.dot_general(
        w_ref[...], x_ref[...],
        (((1,), (1,)), ((), ())),
        preferred_element_type=jnp.float32,
    ) + b_ref[...]


_SC_INFO = plsc.get_sparse_core_info()
_NC = _SC_INFO.num_cores
_NW = _NC * _SC_INFO.num_subcores
_LANES = _SC_INFO.num_lanes


def _route_sc_body(tpw, logits_hbm, out_hbm, idx_hbm, lbuf, obuf, ibuf):
    wid = lax.axis_index("s") * _NC + lax.axis_index("c")
    base = wid * tpw
    pltpu.sync_copy(logits_hbm.at[:, pl.ds(base, tpw)], lbuf)

    neg_inf = jnp.full((_LANES,), -jnp.inf, dtype=jnp.float32)
    zero = jnp.zeros((_LANES,), dtype=jnp.float32)

    for g in range(tpw // _LANES):
        sl = pl.ds(g * _LANES, _LANES)
        cols = [lbuf[e, sl] for e in range(_EXPERTS)]

        v1 = cols[0]
        idx1 = jnp.zeros((_LANES,), dtype=jnp.int32)
        for e in range(1, _EXPERTS):
            m = cols[e] > v1
            v1 = jnp.where(m, cols[e], v1)
            idx1 = jnp.where(m, jnp.full((_LANES,), e, jnp.int32), idx1)

        v2 = jnp.where(idx1 == 0, neg_inf, cols[0])
        idx2 = jnp.zeros((_LANES,), dtype=jnp.int32)
        for e in range(1, _EXPERTS):
            m = (cols[e] > v2) & (idx1 != e)
            v2 = jnp.where(m, cols[e], v2)
            idx2 = jnp.where(m, jnp.full((_LANES,), e, jnp.int32), idx2)

        t = jnp.exp(v2 - v1)
        denom = 1.0 + t
        p1 = 1.0 / denom
        p2 = t / denom

        for e in range(_EXPERTS):
            obuf[e, sl] = (jnp.where(idx1 == e, p1, zero)
                           + jnp.where(idx2 == e, p2, zero))
        ibuf[0, sl] = idx1
        ibuf[1, sl] = idx2

    pltpu.sync_copy(obuf, out_hbm.at[:, pl.ds(base, tpw)])
    pltpu.sync_copy(ibuf, idx_hbm.at[:, pl.ds(base, tpw)])


def _finalize_kernel(outt_ref, idxt_ref, out_ref, idx_ref):
    out_ref[...] = outt_ref[...].T
    idx_ref[...] = idxt_ref[...].T


def kernel(x, route_W, route_b, noise_W, noise_b):
    del noise_W, noise_b
    tokens = x.shape[0]
    n_embd = x.shape[1]
    bcol = route_b.reshape(_EXPERTS, 1)
    ctok = tokens // _CHUNKS
    tpw = ctok // _NW

    mesh = plsc.VectorSubcoreMesh(core_axis_name="c", subcore_axis_name="s",
                                  num_cores=_NC)
    route = functools.partial(
        pl.kernel,
        out_type=[
            jax.ShapeDtypeStruct((_EXPERTS, ctok), jnp.float32),
            jax.ShapeDtypeStruct((_TOP_K, ctok), jnp.int32),
        ],
        mesh=mesh,
        scratch_types=[
            pltpu.VMEM((_EXPERTS, tpw), jnp.float32),
            pltpu.VMEM((_EXPERTS, tpw), jnp.float32),
            pltpu.VMEM((_TOP_K, tpw), jnp.int32),
        ],
    )(functools.partial(_route_sc_body, tpw))

    def _mm(c):
        blk0 = c * (ctok // _BLK)
        return pl.pallas_call(
            _logits_t_kernel,
            grid=(ctok // _BLK,),
            in_specs=[
                pl.BlockSpec((_BLK, n_embd), lambda i, b=blk0: (b + i, 0)),
                pl.BlockSpec((_EXPERTS, n_embd), lambda i: (0, 0)),
                pl.BlockSpec((_EXPERTS, 1), lambda i: (0, 0)),
            ],
            out_specs=pl.BlockSpec((_EXPERTS, _BLK), lambda i: (0, i)),
            out_shape=jax.ShapeDtypeStruct((_EXPERTS, ctok), jnp.float32),
            compiler_params=pltpu.CompilerParams(
                dimension_semantics=("arbitrary",),
            ),
        )(x, route_W, bcol)

    def _fin(out_t, idx_t):
        return pl.pallas_call(
            _finalize_kernel,
            grid=(ctok // _BLK,),
            in_specs=[
                pl.BlockSpec((_EXPERTS, _BLK), lambda i: (0, i)),
                pl.BlockSpec((_TOP_K, _BLK), lambda i: (0, i)),
            ],
            out_specs=[
                pl.BlockSpec((_BLK, _EXPERTS), lambda i: (i, 0)),
                pl.BlockSpec((_BLK, _TOP_K), lambda i: (i, 0)),
            ],
            out_shape=[
                jax.ShapeDtypeStruct((ctok, _EXPERTS), jnp.float32),
                jax.ShapeDtypeStruct((ctok, _TOP_K), jnp.int32),
            ],
            compiler_params=pltpu.CompilerParams(
                dimension_semantics=("arbitrary",),
            ),
        )(out_t, idx_t)

    logits = [_mm(0)]
    routed = []
    for c in range(_CHUNKS):
        routed.append(route(logits[c]))
        if c + 1 < _CHUNKS:
            logits.append(_mm(c + 1))
    fins = [_fin(out_t, idx_t) for out_t, idx_t in routed]
    if _CHUNKS == 1:
        return fins[0]
    return (jnp.concatenate([f[0] for f in fins], axis=0),
            jnp.concatenate([f[1] for f in fins], axis=0))
